# baseline (speedup 1.0000x reference)
import functools

import jax
import jax.numpy as jnp
from jax import lax
from jax.experimental import pallas as pl
from jax.experimental.pallas import tpu as pltpu
from jax.experimental.pallas import tpu_sc as plsc

VOCAB = 100000
EMBED = 64
MAX_LEN = 200
B = 4096
L = 200

NC, NS, LANES = 2, 16, 16
NW = NC * NS
TOKENS = B * L
TOK_PER_W = TOKENS // NW
CHUNK = 512
N_CHUNKS = TOK_PER_W // CHUNK
N_GROUPS = CHUNK // LANES
GATHER_SPLIT = 128


def _body(seq_hbm, table_hbm, pos_hbm, w_hbm, b_hbm, out_hbm,
          idx_v, rows_v, vT, pos_vm, w_vm, b_vm, sem):
    wid = lax.axis_index("s") * NC + lax.axis_index("c")

    pltpu.sync_copy(pos_hbm, pos_vm)
    pltpu.sync_copy(w_hbm, w_vm)
    pltpu.sync_copy(b_hbm, b_vm)

    iota16 = lax.iota(jnp.int32, LANES)
    inv_e = jnp.float32(1.0 / EMBED)

    def chunk_body(c, _):
        base = wid * TOK_PER_W + c * CHUNK
        pltpu.sync_copy(seq_hbm.at[pl.ds(base, CHUNK)], idx_v)
        cps = [
            pltpu.async_copy(
                table_hbm.at[idx_v.at[pl.ds(j * GATHER_SPLIT, GATHER_SPLIT)]],
                rows_v.at[pl.ds(j * GATHER_SPLIT, GATHER_SPLIT)],
                sem)
            for j in range(CHUNK // GATHER_SPLIT)
        ]
        for cp in cps:
            cp.wait()

        def group_body(g, _):
            tok0 = g * LANES
            tokv = iota16 + tok0
            lvec = lax.rem(base + tokv, MAX_LEN)
            iv = idx_v[pl.ds(tok0, LANES)]
            m = jnp.where(iv != 0, jnp.float32(1.0), jnp.float32(0.0))

            def d_body(dd, carry):
                s, ss = carry
                dv = jnp.full((LANES,), dd, jnp.int32)
                g_d = plsc.load_gather(rows_v, [tokv, dv])
                p_d = plsc.load_gather(pos_vm, [lvec, dv])
                v_d = g_d * m + p_d
                vT[pl.ds(dd * LANES, LANES)] = v_d
                return s + v_d, ss + v_d * v_d

            zero = jnp.zeros((LANES,), jnp.float32)
            s, ss = lax.fori_loop(0, EMBED, d_body, (zero, zero))
            mean = s * inv_e
            var = jnp.maximum(ss * inv_e - mean * mean, 0.0) + jnp.float32(1e-12)
            ii = jnp.int32(0x5F3759DF) - lax.shift_right_arithmetic(
                plsc.bitcast(var, jnp.int32), 1)
            y = plsc.bitcast(ii, jnp.float32)
            for _ in range(3):
                y = y * (jnp.float32(1.5) - jnp.float32(0.5) * var * y * y)
            rs = y

            def d2_body(dd, _):
                dv = jnp.full((LANES,), dd, jnp.int32)
                v_d = vT[pl.ds(dd * LANES, LANES)]
                wv = plsc.load_gather(w_vm, [dv])
                bv = plsc.load_gather(b_vm, [dv])
                o_d = (v_d - mean) * rs * wv + bv
                plsc.store_scatter(rows_v, [tokv, dv], o_d)
                return 0

            lax.fori_loop(0, EMBED, d2_body, 0)
            return 0

        lax.fori_loop(0, N_GROUPS, group_body, 0)
        pltpu.sync_copy(rows_v, out_hbm.at[pl.ds(base, CHUNK)])
        return 0

    lax.fori_loop(0, N_CHUNKS, chunk_body, 0)


@jax.jit
def kernel(seq, token_table, pos_table, ln_weight, ln_bias):
    seq_flat = seq.reshape(-1).astype(jnp.int32)
    mesh = plsc.VectorSubcoreMesh(core_axis_name="c", subcore_axis_name="s")
    out = pl.kernel(
        _body,
        out_type=jax.ShapeDtypeStruct((TOKENS, EMBED), jnp.float32),
        mesh=mesh,
        compiler_params=pltpu.CompilerParams(
            use_tc_tiling_on_sc=False,
            needs_layout_passes=False,
        ),
        scratch_types=[
            pltpu.VMEM((CHUNK,), jnp.int32),
            pltpu.VMEM((CHUNK, EMBED), jnp.float32),
            pltpu.VMEM((LANES * EMBED,), jnp.float32),
            pltpu.VMEM((MAX_LEN, EMBED), jnp.float32),
            pltpu.VMEM((EMBED,), jnp.float32),
            pltpu.VMEM((EMBED,), jnp.float32),
            pltpu.SemaphoreType.DMA,
        ],
    )(seq_flat, token_table, pos_table, ln_weight, ln_bias)
    return out.reshape(B, L, EMBED)


# speedup vs baseline: 1.0392x; 1.0392x over previous
import jax
import jax.numpy as jnp
from jax import lax
from jax.experimental import pallas as pl
from jax.experimental.pallas import tpu as pltpu
from jax.experimental.pallas import tpu_sc as plsc

VOCAB = 100000
EMBED = 64
MAX_LEN = 200
B = 4096
L = 200

NC, NS, LANES = 2, 16, 16
NW = NC * NS
TOKENS = B * L
TOK_PER_W = TOKENS // NW
CHUNK = 512
N_CHUNKS = TOK_PER_W // CHUNK
N_GROUPS = CHUNK // LANES
GATHER_SPLIT = 128
UNROLL = 8


def _body(seq_hbm, table_hbm, pos_hbm, w_hbm, b_hbm, out_hbm,
          idx0, idx1, rows0, rows1, vT, pos_vm, w_vm, b_vm,
          gsem0, gsem1, osem0, osem1):
    wid = lax.axis_index("s") * NC + lax.axis_index("c")

    pltpu.sync_copy(pos_hbm, pos_vm)
    pltpu.sync_copy(w_hbm, w_vm)
    pltpu.sync_copy(b_hbm, b_vm)

    iota16 = lax.iota(jnp.int32, LANES)
    inv_e = jnp.float32(1.0 / EMBED)
    bufs = ((idx0, rows0, gsem0, osem0), (idx1, rows1, gsem1, osem1))

    def base_of(c):
        return wid * TOK_PER_W + c * CHUNK

    def issue_gather(c, idx_v, rows_v, gsem):
        pltpu.sync_copy(seq_hbm.at[pl.ds(base_of(c), CHUNK)], idx_v)
        for j in range(CHUNK // GATHER_SPLIT):
            pltpu.async_copy(
                table_hbm.at[idx_v.at[pl.ds(j * GATHER_SPLIT, GATHER_SPLIT)]],
                rows_v.at[pl.ds(j * GATHER_SPLIT, GATHER_SPLIT)],
                gsem)

    def wait_gather(idx_v, rows_v, gsem):
        for j in range(CHUNK // GATHER_SPLIT):
            pltpu.make_async_copy(
                table_hbm.at[idx_v.at[pl.ds(j * GATHER_SPLIT, GATHER_SPLIT)]],
                rows_v.at[pl.ds(j * GATHER_SPLIT, GATHER_SPLIT)],
                gsem).wait()

    def issue_out(c, rows_v, osem):
        pltpu.async_copy(rows_v, out_hbm.at[pl.ds(base_of(c), CHUNK)], osem)

    def wait_out(c, rows_v, osem):
        pltpu.make_async_copy(
            rows_v, out_hbm.at[pl.ds(base_of(c), CHUNK)], osem).wait()

    def compute(c, idx_v, rows_v):
        base = base_of(c)

        def group_body(g, _):
            tok0 = g * LANES
            tokv = iota16 + tok0
            lvec = lax.rem(base + tokv, MAX_LEN)
            iv = idx_v[pl.ds(tok0, LANES)]
            m = jnp.where(iv != 0, jnp.float32(1.0), jnp.float32(0.0))

            def d_body(i, carry):
                s, ss = carry
                d0 = i * UNROLL
                for k in range(UNROLL):
                    dd = d0 + k
                    dv = jnp.full((LANES,), dd, jnp.int32)
                    g_d = plsc.load_gather(rows_v, [tokv, dv])
                    p_d = plsc.load_gather(pos_vm, [lvec, dv])
                    v_d = g_d * m + p_d
                    vT[pl.ds(dd * LANES, LANES)] = v_d
                    s = s + v_d
                    ss = ss + v_d * v_d
                return s, ss

            zero = jnp.zeros((LANES,), jnp.float32)
            s, ss = lax.fori_loop(0, EMBED // UNROLL, d_body, (zero, zero))
            mean = s * inv_e
            var = jnp.maximum(ss * inv_e - mean * mean, 0.0) + jnp.float32(1e-12)
            ii = jnp.int32(0x5F3759DF) - lax.shift_right_arithmetic(
                plsc.bitcast(var, jnp.int32), 1)
            y = plsc.bitcast(ii, jnp.float32)
            for _ in range(3):
                y = y * (jnp.float32(1.5) - jnp.float32(0.5) * var * y * y)
            rs = y

            def d2_body(i, _):
                d0 = i * UNROLL
                for k in range(UNROLL):
                    dd = d0 + k
                    dv = jnp.full((LANES,), dd, jnp.int32)
                    v_d = vT[pl.ds(dd * LANES, LANES)]
                    wv = plsc.load_gather(w_vm, [dv])
                    bv = plsc.load_gather(b_vm, [dv])
                    o_d = (v_d - mean) * rs * wv + bv
                    plsc.store_scatter(rows_v, [tokv, dv], o_d)
                return 0

            lax.fori_loop(0, EMBED // UNROLL, d2_body, 0)
            return 0

        lax.fori_loop(0, N_GROUPS, group_body, 0)

    issue_gather(0, *bufs[0][:3])

    def pair_body(t, _):
        for b in range(2):
            c = 2 * t + b
            cur_idx, cur_rows, cur_gsem, cur_osem = bufs[b]
            nxt_idx, nxt_rows, nxt_gsem, nxt_osem = bufs[1 - b]

            @pl.when(c + 1 < N_CHUNKS)
            def _prefetch():
                @pl.when(c >= 1)
                def _drain_prev():
                    wait_out(c - 1, nxt_rows, nxt_osem)
                issue_gather(c + 1, nxt_idx, nxt_rows, nxt_gsem)

            wait_gather(cur_idx, cur_rows, cur_gsem)
            compute(c, cur_idx, cur_rows)
            issue_out(c, cur_rows, cur_osem)
        return 0

    lax.fori_loop(0, N_CHUNKS // 2, pair_body, 0)
    wait_out(N_CHUNKS - 2, rows0, osem0)
    wait_out(N_CHUNKS - 1, rows1, osem1)


@jax.jit
def kernel(seq, token_table, pos_table, ln_weight, ln_bias):
    seq_flat = seq.reshape(-1).astype(jnp.int32)
    mesh = plsc.VectorSubcoreMesh(core_axis_name="c", subcore_axis_name="s")
    out = pl.kernel(
        _body,
        out_type=jax.ShapeDtypeStruct((TOKENS, EMBED), jnp.float32),
        mesh=mesh,
        compiler_params=pltpu.CompilerParams(
            use_tc_tiling_on_sc=False,
            needs_layout_passes=False,
        ),
        scratch_types=[
            pltpu.VMEM((CHUNK,), jnp.int32),
            pltpu.VMEM((CHUNK,), jnp.int32),
            pltpu.VMEM((CHUNK, EMBED), jnp.float32),
            pltpu.VMEM((CHUNK, EMBED), jnp.float32),
            pltpu.VMEM((LANES * EMBED,), jnp.float32),
            pltpu.VMEM((MAX_LEN, EMBED), jnp.float32),
            pltpu.VMEM((EMBED,), jnp.float32),
            pltpu.VMEM((EMBED,), jnp.float32),
            pltpu.SemaphoreType.DMA,
            pltpu.SemaphoreType.DMA,
            pltpu.SemaphoreType.DMA,
            pltpu.SemaphoreType.DMA,
        ],
    )(seq_flat, token_table, pos_table, ln_weight, ln_bias)
    return out.reshape(B, L, EMBED)


# speedup vs baseline: 1.2713x; 1.2234x over previous
import jax
import jax.numpy as jnp
from jax import lax
from jax.experimental import pallas as pl
from jax.experimental.pallas import tpu as pltpu
from jax.experimental.pallas import tpu_sc as plsc

VOCAB = 100000
EMBED = 64
MAX_LEN = 200
B = 4096
L = 200

NC, NS, LANES = 2, 16, 16
NW = NC * NS
TOKENS = B * L
SEQ_PER_CHUNK = 2
CHUNK = SEQ_PER_CHUNK * L
TOK_PER_W = TOKENS // NW
N_CHUNKS = TOK_PER_W // CHUNK
N_GROUPS = CHUNK // LANES
POS_T = 2 * L
UNROLL = 8
SPLITS = ((0, 128), (128, 72), (200, 128), (328, 72))


def _body(seq_hbm, table_hbm, posT_hbm, w_hbm, b_hbm, out_hbm,
          idx0, idx1, rows0, rows1, vT, posT_vm, w_vm, b_vm,
          gsem0, gsem1, osem0, osem1):
    wid = lax.axis_index("s") * NC + lax.axis_index("c")

    pltpu.sync_copy(posT_hbm, posT_vm)
    pltpu.sync_copy(w_hbm, w_vm)
    pltpu.sync_copy(b_hbm, b_vm)

    iota16 = lax.iota(jnp.int32, LANES)
    inv_e = jnp.float32(1.0 / EMBED)
    bufs = ((idx0, rows0, gsem0, osem0), (idx1, rows1, gsem1, osem1))

    def base_of(c):
        return wid * TOK_PER_W + c * CHUNK

    def issue_gather(c, idx_v, rows_v, gsem):
        pltpu.sync_copy(seq_hbm.at[pl.ds(base_of(c), CHUNK)], idx_v)
        for (start, size) in SPLITS:
            pltpu.async_copy(
                table_hbm.at[idx_v.at[pl.ds(start, size)]],
                rows_v.at[pl.ds(start, size)],
                gsem)

    def wait_gather(idx_v, rows_v, gsem):
        for (start, size) in SPLITS:
            pltpu.make_async_copy(
                table_hbm.at[idx_v.at[pl.ds(start, size)]],
                rows_v.at[pl.ds(start, size)],
                gsem).wait()

    def issue_out(c, rows_v, osem):
        bb = (base_of(c)) // L
        for q in range(SEQ_PER_CHUNK):
            pltpu.async_copy(rows_v.at[pl.ds(q * L, L)],
                             out_hbm.at[bb + q], osem)

    def wait_out(c, rows_v, osem):
        bb = (base_of(c)) // L
        for q in range(SEQ_PER_CHUNK):
            pltpu.make_async_copy(rows_v.at[pl.ds(q * L, L)],
                                  out_hbm.at[bb + q], osem).wait()

    def compute(idx_v, rows_v):
        def group_body(g, _):
            tok0 = g * LANES
            tokv = iota16 + tok0
            l0 = lax.rem(tok0, L)
            iv = idx_v[pl.ds(tok0, LANES)]
            m = jnp.where(iv != 0, jnp.float32(1.0), jnp.float32(0.0))

            def d_body(i, carry):
                s, ss = carry
                d0 = i * UNROLL
                vs, qs = [], []
                for k in range(UNROLL):
                    dd = d0 + k
                    dv = jnp.full((LANES,), dd, jnp.int32)
                    g_d = plsc.load_gather(rows_v, [tokv, dv])
                    p_d = posT_vm[pl.ds(dd * POS_T + l0, LANES)]
                    v_d = g_d * m + p_d
                    vT[pl.ds(dd * LANES, LANES)] = v_d
                    vs.append(v_d)
                    qs.append(v_d * v_d)
                while len(vs) > 1:
                    vs = [a + b for a, b in zip(vs[::2], vs[1::2])]
                    qs = [a + b for a, b in zip(qs[::2], qs[1::2])]
                return s + vs[0], ss + qs[0]

            zero = jnp.zeros((LANES,), jnp.float32)
            s, ss = lax.fori_loop(0, EMBED // UNROLL, d_body, (zero, zero))
            mean = s * inv_e
            var = jnp.maximum(ss * inv_e - mean * mean, 0.0) + jnp.float32(1e-12)
            ii = jnp.int32(0x5F3759DF) - lax.shift_right_arithmetic(
                plsc.bitcast(var, jnp.int32), 1)
            y = plsc.bitcast(ii, jnp.float32)
            for _ in range(3):
                y = y * (jnp.float32(1.5) - jnp.float32(0.5) * var * y * y)
            rs = y

            def d2_body(i, _):
                d0 = i * UNROLL
                for k in range(UNROLL):
                    dd = d0 + k
                    dv = jnp.full((LANES,), dd, jnp.int32)
                    v_d = vT[pl.ds(dd * LANES, LANES)]
                    wv = w_vm[pl.ds(dd * LANES, LANES)]
                    bv = b_vm[pl.ds(dd * LANES, LANES)]
                    o_d = (v_d - mean) * rs * wv + bv
                    plsc.store_scatter(rows_v, [tokv, dv], o_d)
                return 0

            lax.fori_loop(0, EMBED // UNROLL, d2_body, 0)
            return 0

        lax.fori_loop(0, N_GROUPS, group_body, 0)

    issue_gather(0, *bufs[0][:3])

    def pair_body(t, _):
        for b in range(2):
            c = 2 * t + b
            cur_idx, cur_rows, cur_gsem, cur_osem = bufs[b]
            nxt_idx, nxt_rows, nxt_gsem, nxt_osem = bufs[1 - b]

            @pl.when(c + 1 < N_CHUNKS)
            def _prefetch():
                @pl.when(c >= 1)
                def _drain_prev():
                    wait_out(c - 1, nxt_rows, nxt_osem)
                issue_gather(c + 1, nxt_idx, nxt_rows, nxt_gsem)

            wait_gather(cur_idx, cur_rows, cur_gsem)
            compute(cur_idx, cur_rows)
            issue_out(c, cur_rows, cur_osem)
        return 0

    lax.fori_loop(0, N_CHUNKS // 2, pair_body, 0)
    wait_out(N_CHUNKS - 2, rows0, osem0)
    wait_out(N_CHUNKS - 1, rows1, osem1)


@jax.jit
def kernel(seq, token_table, pos_table, ln_weight, ln_bias):
    seq_flat = seq.reshape(-1).astype(jnp.int32)
    posT = jnp.tile(pos_table.T, (1, 2)).reshape(-1)
    w_splat = jnp.repeat(ln_weight, LANES)
    b_splat = jnp.repeat(ln_bias, LANES)
    mesh = plsc.VectorSubcoreMesh(core_axis_name="c", subcore_axis_name="s")
    out = pl.kernel(
        _body,
        out_type=jax.ShapeDtypeStruct((B, L, EMBED), jnp.float32),
        mesh=mesh,
        compiler_params=pltpu.CompilerParams(
            use_tc_tiling_on_sc=False,
            needs_layout_passes=False,
        ),
        scratch_types=[
            pltpu.VMEM((CHUNK,), jnp.int32),
            pltpu.VMEM((CHUNK,), jnp.int32),
            pltpu.VMEM((CHUNK, EMBED), jnp.float32),
            pltpu.VMEM((CHUNK, EMBED), jnp.float32),
            pltpu.VMEM((LANES * EMBED,), jnp.float32),
            pltpu.VMEM((EMBED * POS_T,), jnp.float32),
            pltpu.VMEM((EMBED * LANES,), jnp.float32),
            pltpu.VMEM((EMBED * LANES,), jnp.float32),
            pltpu.SemaphoreType.DMA,
            pltpu.SemaphoreType.DMA,
            pltpu.SemaphoreType.DMA,
            pltpu.SemaphoreType.DMA,
        ],
    )(seq_flat, token_table, posT, w_splat, b_splat)
    return out


# speedup vs baseline: 1.9022x; 1.4962x over previous
import jax
import jax.numpy as jnp
from jax import lax
from jax.experimental import pallas as pl
from jax.experimental.pallas import tpu as pltpu
from jax.experimental.pallas import tpu_sc as plsc

VOCAB = 100000
EMBED = 64
MAX_LEN = 200
B = 4096
L = 200

NC, NS, LANES = 2, 16, 16
NW = NC * NS
TOKENS = B * L
SEQ_PER_CHUNK = 2
CHUNK = SEQ_PER_CHUNK * L
TOK_PER_W = TOKENS // NW
N_CHUNKS = TOK_PER_W // CHUNK
N_GROUPS = CHUNK // LANES
POS_T = 2 * L
UNROLL = 8
SPLITS = ((0, 128), (128, 72), (200, 128), (328, 72))


def _body(seq_hbm, table_hbm, posT_hbm, w_hbm, b_hbm, out_hbm,
          idx0, idx1, rows0, rows1, vT, posT_vm, w_vm, b_vm,
          gsem0, gsem1, osem0, osem1):
    wid = lax.axis_index("s") * NC + lax.axis_index("c")

    pltpu.sync_copy(posT_hbm, posT_vm)
    pltpu.sync_copy(w_hbm, w_vm)
    pltpu.sync_copy(b_hbm, b_vm)

    iota16 = lax.iota(jnp.int32, LANES)
    inv_e = jnp.float32(1.0 / EMBED)
    bufs = ((idx0, rows0, gsem0, osem0), (idx1, rows1, gsem1, osem1))

    def base_of(c):
        return wid * TOK_PER_W + c * CHUNK

    def issue_gather(c, idx_v, rows_v, gsem):
        pltpu.sync_copy(seq_hbm.at[pl.ds(base_of(c), CHUNK)], idx_v)
        for (start, size) in SPLITS:
            pltpu.async_copy(
                table_hbm.at[idx_v.at[pl.ds(start, size)]],
                rows_v.at[pl.ds(start, size)],
                gsem)

    def wait_gather(idx_v, rows_v, gsem):
        for (start, size) in SPLITS:
            pltpu.make_async_copy(
                table_hbm.at[idx_v.at[pl.ds(start, size)]],
                rows_v.at[pl.ds(start, size)],
                gsem).wait()

    def issue_out(c, rows_v, osem):
        bb = (base_of(c)) // L
        for q in range(SEQ_PER_CHUNK):
            pltpu.async_copy(rows_v.at[pl.ds(q * L, L)],
                             out_hbm.at[bb + q], osem)

    def wait_out(c, rows_v, osem):
        bb = (base_of(c)) // L
        for q in range(SEQ_PER_CHUNK):
            pltpu.make_async_copy(rows_v.at[pl.ds(q * L, L)],
                                  out_hbm.at[bb + q], osem).wait()

    def compute(idx_v, rows_v):
        def group_body(g, _):
            tok0 = g * LANES
            tokv = iota16 + tok0
            l0 = lax.rem(tok0, L)
            iv = idx_v[pl.ds(tok0, LANES)]
            m = jnp.where(iv != 0, jnp.float32(1.0), jnp.float32(0.0))

            zero = jnp.zeros((LANES,), jnp.float32)

            @plsc.parallel_loop(0, EMBED // UNROLL, 1, unroll=2,
                                carry=(zero, zero))
            def d_carry(i, carry):
                s, ss = carry
                d0 = i * UNROLL
                vs, qs = [], []
                for k in range(UNROLL):
                    dd = d0 + k
                    dv = jnp.full((LANES,), dd, jnp.int32)
                    g_d = plsc.load_gather(rows_v, [tokv, dv])
                    p_d = posT_vm[pl.ds(dd * POS_T + l0, LANES)]
                    v_d = g_d * m + p_d
                    vT[pl.ds(dd * LANES, LANES)] = v_d
                    vs.append(v_d)
                    qs.append(v_d * v_d)
                while len(vs) > 1:
                    vs = [a + b for a, b in zip(vs[::2], vs[1::2])]
                    qs = [a + b for a, b in zip(qs[::2], qs[1::2])]
                return s + vs[0], ss + qs[0]

            s, ss = d_carry
            mean = s * inv_e
            var = jnp.maximum(ss * inv_e - mean * mean, 0.0) + jnp.float32(1e-12)
            ii = jnp.int32(0x5F3759DF) - lax.shift_right_arithmetic(
                plsc.bitcast(var, jnp.int32), 1)
            y = plsc.bitcast(ii, jnp.float32)
            for _ in range(3):
                y = y * (jnp.float32(1.5) - jnp.float32(0.5) * var * y * y)
            rs = y

            @plsc.parallel_loop(0, EMBED // UNROLL, 1, unroll=2)
            def _d2(i):
                d0 = i * UNROLL
                for k in range(UNROLL):
                    dd = d0 + k
                    dv = jnp.full((LANES,), dd, jnp.int32)
                    v_d = vT[pl.ds(dd * LANES, LANES)]
                    wv = w_vm[pl.ds(dd * LANES, LANES)]
                    bv = b_vm[pl.ds(dd * LANES, LANES)]
                    o_d = (v_d - mean) * rs * wv + bv
                    plsc.store_scatter(rows_v, [tokv, dv], o_d)
            return 0

        lax.fori_loop(0, N_GROUPS, group_body, 0)

    issue_gather(0, *bufs[0][:3])

    def pair_body(t, _):
        for b in range(2):
            c = 2 * t + b
            cur_idx, cur_rows, cur_gsem, cur_osem = bufs[b]
            nxt_idx, nxt_rows, nxt_gsem, nxt_osem = bufs[1 - b]

            @pl.when(c + 1 < N_CHUNKS)
            def _prefetch():
                @pl.when(c >= 1)
                def _drain_prev():
                    wait_out(c - 1, nxt_rows, nxt_osem)
                issue_gather(c + 1, nxt_idx, nxt_rows, nxt_gsem)

            wait_gather(cur_idx, cur_rows, cur_gsem)
            compute(cur_idx, cur_rows)
            issue_out(c, cur_rows, cur_osem)
        return 0

    lax.fori_loop(0, N_CHUNKS // 2, pair_body, 0)
    wait_out(N_CHUNKS - 2, rows0, osem0)
    wait_out(N_CHUNKS - 1, rows1, osem1)


@jax.jit
def kernel(seq, token_table, pos_table, ln_weight, ln_bias):
    seq_flat = seq.reshape(-1).astype(jnp.int32)
    posT = jnp.tile(pos_table.T, (1, 2)).reshape(-1)
    w_splat = jnp.repeat(ln_weight, LANES)
    b_splat = jnp.repeat(ln_bias, LANES)
    mesh = plsc.VectorSubcoreMesh(core_axis_name="c", subcore_axis_name="s")
    out = pl.kernel(
        _body,
        out_type=jax.ShapeDtypeStruct((B, L, EMBED), jnp.float32),
        mesh=mesh,
        compiler_params=pltpu.CompilerParams(
            use_tc_tiling_on_sc=False,
            needs_layout_passes=False,
        ),
        scratch_types=[
            pltpu.VMEM((CHUNK,), jnp.int32),
            pltpu.VMEM((CHUNK,), jnp.int32),
            pltpu.VMEM((CHUNK, EMBED), jnp.float32),
            pltpu.VMEM((CHUNK, EMBED), jnp.float32),
            pltpu.VMEM((LANES * EMBED,), jnp.float32),
            pltpu.VMEM((EMBED * POS_T,), jnp.float32),
            pltpu.VMEM((EMBED * LANES,), jnp.float32),
            pltpu.VMEM((EMBED * LANES,), jnp.float32),
            pltpu.SemaphoreType.DMA,
            pltpu.SemaphoreType.DMA,
            pltpu.SemaphoreType.DMA,
            pltpu.SemaphoreType.DMA,
        ],
    )(seq_flat, token_table, posT, w_splat, b_splat)
    return out
